# baseline (speedup 1.0000x reference)
import functools

import jax
import jax.numpy as jnp
from jax import lax
from jax.experimental import pallas as pl
from jax.experimental.pallas import tpu as pltpu, tpu_sc as plsc

_N = 10000
_E = 320000
_D = 128
_H = 16

_NC = 2
_NS = 16
_NW = _NC * _NS
_CH = 512
_KCH = 20
_EPW = _KCH * _CH
_PADE = _NW * _EPW - _E
_NPAD = 10112
_ZR = _NPAD // _NS

_mesh = plsc.VectorSubcoreMesh(core_axis_name="c", subcore_axis_name="s")
_sc_params = pltpu.CompilerParams(use_tc_tiling_on_sc=False)


@functools.partial(
    pl.kernel,
    mesh=_mesh,
    compiler_params=_sc_params,
    out_type=jax.ShapeDtypeStruct((_NC, _NPAD, _H), jnp.float32),
    scratch_types=[
        pltpu.VMEM((_KCH, _CH), jnp.int32),
        pltpu.VMEM((_CH, _H), jnp.float32),
        pltpu.VMEM((_ZR, _H), jnp.float32),
        pltpu.VMEM_SHARED((_NPAD, _H), jnp.float32),
        pltpu.SemaphoreType.DMA,
    ],
)
def _sc_deg(dst_hbm, out_hbm, idx_v, ones_v, zbuf_v, acc_sh, sem):
    c = lax.axis_index("c")
    s = lax.axis_index("s")
    wid = s * _NC + c
    pltpu.sync_copy(dst_hbm.at[wid], idx_v)

    def _fill(i, carry):
        ones_v[i, :] = jnp.full((_H,), 1.0, jnp.float32)
        return carry

    lax.fori_loop(0, _CH, _fill, 0)

    def _zero(i, carry):
        zbuf_v[i, :] = jnp.zeros((_H,), jnp.float32)
        return carry

    lax.fori_loop(0, _ZR, _zero, 0)
    pltpu.sync_copy(zbuf_v, acc_sh.at[pl.ds(s * _ZR, _ZR)])
    plsc.subcore_barrier()

    def _scat(k, carry):
        pltpu.async_copy(ones_v, acc_sh.at[idx_v.at[k]], sem, add=True)
        return carry

    lax.fori_loop(0, _KCH, _scat, 0)

    def _drain(k, carry):
        pltpu.make_async_copy(ones_v, acc_sh.at[idx_v.at[k]], sem).wait()
        return carry

    lax.fori_loop(0, _KCH, _drain, 0)
    plsc.subcore_barrier()
    pltpu.sync_copy(acc_sh.at[pl.ds(s * _ZR, _ZR)],
                    out_hbm.at[c, pl.ds(s * _ZR, _ZR)])


@functools.partial(
    pl.kernel,
    mesh=_mesh,
    compiler_params=_sc_params,
    out_type=jax.ShapeDtypeStruct((_NC, _NPAD, _H), jnp.float32),
    scratch_types=[
        pltpu.VMEM((_KCH, _CH), jnp.int32),
        pltpu.VMEM((_KCH, _CH), jnp.int32),
        pltpu.VMEM((2, _CH, _H), jnp.float32),
        pltpu.VMEM((_ZR, _H), jnp.float32),
        pltpu.VMEM_SHARED((_NPAD, _H), jnp.float32),
        pltpu.SemaphoreType.DMA,
        pltpu.SemaphoreType.DMA,
    ],
)
def _sc_edge_sum(tab_hbm, src_hbm, dst_hbm, out_hbm,
                 src_v, dst_v, rows_v, zbuf_v, acc_sh, sem_g, sem_s):
    c = lax.axis_index("c")
    s = lax.axis_index("s")
    wid = s * _NC + c
    pltpu.sync_copy(src_hbm.at[wid], src_v)
    pltpu.sync_copy(dst_hbm.at[wid], dst_v)

    def _zero(i, carry):
        zbuf_v[i, :] = jnp.zeros((_H,), jnp.float32)
        return carry

    lax.fori_loop(0, _ZR, _zero, 0)
    pltpu.sync_copy(zbuf_v, acc_sh.at[pl.ds(s * _ZR, _ZR)])
    plsc.subcore_barrier()

    pltpu.async_copy(tab_hbm.at[src_v.at[0]], rows_v.at[0], sem_g)

    def _body(k, carry):
        b = lax.rem(k, 2)

        @pl.when(k >= 1)
        def _wait_prev_scatter():
            pltpu.make_async_copy(rows_v.at[1 - b],
                                  acc_sh.at[dst_v.at[k - 1]], sem_s).wait()

        @pl.when(k + 1 < _KCH)
        def _issue_next_gather():
            pltpu.async_copy(tab_hbm.at[src_v.at[k + 1]],
                             rows_v.at[1 - b], sem_g)

        pltpu.make_async_copy(tab_hbm.at[src_v.at[k]],
                              rows_v.at[b], sem_g).wait()
        pltpu.async_copy(rows_v.at[b], acc_sh.at[dst_v.at[k]], sem_s,
                         add=True)
        return carry

    lax.fori_loop(0, _KCH, _body, 0)
    pltpu.make_async_copy(rows_v.at[(_KCH - 1) % 2],
                          acc_sh.at[dst_v.at[_KCH - 1]], sem_s).wait()
    plsc.subcore_barrier()
    pltpu.sync_copy(acc_sh.at[pl.ds(s * _ZR, _ZR)],
                    out_hbm.at[c, pl.ds(s * _ZR, _ZR)])


def _tc1_body(x_ref, w_ref, d0_ref, d1_ref, xs_ref, dinv_ref):
    deg = d0_ref[...] + d1_ref[...] + 1.0
    dinv = lax.rsqrt(deg)
    xw = jnp.dot(x_ref[...], w_ref[...], preferred_element_type=jnp.float32)
    xs_ref[...] = xw * dinv
    dinv_ref[...] = dinv


def _tc2_body(s0_ref, s1_ref, xs_ref, dinv_ref, b_ref, w_ref, out_ref):
    dinv = dinv_ref[...]
    h = dinv * (s0_ref[...] + s1_ref[...] + xs_ref[...]) + b_ref[...]
    h = jnp.maximum(h, 0.0)
    out_ref[...] = jnp.dot(h, w_ref[...],
                           preferred_element_type=jnp.float32) * dinv


def _tc3_body(s0_ref, s1_ref, xs_ref, dinv_ref, b_ref, w_ref, bf_ref, out_ref):
    dinv = dinv_ref[...]
    h = dinv * (s0_ref[...] + s1_ref[...] + xs_ref[...]) + b_ref[...]
    h = jnp.maximum(h, 0.0)
    out_ref[...] = jnp.dot(h, w_ref[...],
                           preferred_element_type=jnp.float32) + bf_ref[...]


_tc1 = pl.pallas_call(
    _tc1_body,
    out_shape=(jax.ShapeDtypeStruct((_N, _H), jnp.float32),
               jax.ShapeDtypeStruct((_N, 1), jnp.float32)),
)

_tc2 = pl.pallas_call(
    _tc2_body,
    out_shape=jax.ShapeDtypeStruct((_N, _H), jnp.float32),
)

_tc3 = pl.pallas_call(
    _tc3_body,
    out_shape=jax.ShapeDtypeStruct((_N, 1), jnp.float32),
)


def kernel(x, edge_index, W1, b1, W2, b2, Wfc, bfc):
    ei = edge_index.astype(jnp.int32)
    src = ei[0]
    dst = ei[1]
    pad_src = jnp.zeros((_PADE,), jnp.int32)
    pad_dst = _N + (jnp.arange(_PADE, dtype=jnp.int32) % (_NPAD - _N))
    src_p = jnp.concatenate([src, pad_src]).reshape(_NW, _KCH, _CH)
    dst_p = jnp.concatenate([dst, pad_dst]).reshape(_NW, _KCH, _CH)

    dega = _sc_deg(dst_p)
    d0 = dega[0, :_N, 0:1]
    d1 = dega[1, :_N, 0:1]

    xs1, dinv = _tc1(x, W1, d0, d1)
    s1 = _sc_edge_sum(xs1, src_p, dst_p)
    xs2 = _tc2(s1[0, :_N], s1[1, :_N], xs1, dinv, b1.reshape(1, _H), W2)
    s2 = _sc_edge_sum(xs2, src_p, dst_p)
    out = _tc3(s2[0, :_N], s2[1, :_N], xs2, dinv, b2.reshape(1, _H),
               Wfc, bfc.reshape(1, 1))
    return out


# speedup vs baseline: 1.0316x; 1.0316x over previous
import functools

import jax
import jax.numpy as jnp
from jax import lax
from jax.experimental import pallas as pl
from jax.experimental.pallas import tpu as pltpu, tpu_sc as plsc

_N = 10000
_E = 320000
_D = 128
_H = 16

_NC = 2
_NS = 16
_NW = _NC * _NS
_CH = 512
_KCH = 20
_EPW = _KCH * _CH
_PADE = _NW * _EPW - _E
_NPAD = 10112
_ZR = _NPAD // _NS

_mesh = plsc.VectorSubcoreMesh(core_axis_name="c", subcore_axis_name="s")
_sc_params = pltpu.CompilerParams(use_tc_tiling_on_sc=False)


@functools.partial(
    pl.kernel,
    mesh=_mesh,
    compiler_params=_sc_params,
    out_type=jax.ShapeDtypeStruct((_NC, _NPAD, _H), jnp.float32),
    scratch_types=[
        pltpu.VMEM((_KCH, _CH), jnp.int32),
        pltpu.VMEM((_CH, _H), jnp.float32),
        pltpu.VMEM((_ZR, _H), jnp.float32),
        pltpu.VMEM_SHARED((_NPAD, _H), jnp.float32),
        pltpu.SemaphoreType.DMA,
    ],
)
def _sc_deg(dst_hbm, out_hbm, idx_v, ones_v, zbuf_v, acc_sh, sem):
    c = lax.axis_index("c")
    s = lax.axis_index("s")
    wid = s * _NC + c
    pltpu.sync_copy(dst_hbm.at[wid], idx_v)

    def _fill(i, carry):
        ones_v[i, :] = jnp.full((_H,), 1.0, jnp.float32)
        return carry

    lax.fori_loop(0, _CH, _fill, 0)

    def _zero(i, carry):
        zbuf_v[i, :] = jnp.zeros((_H,), jnp.float32)
        return carry

    lax.fori_loop(0, _ZR, _zero, 0)
    pltpu.sync_copy(zbuf_v, acc_sh.at[pl.ds(s * _ZR, _ZR)])
    plsc.subcore_barrier()

    def _scat(k, carry):
        pltpu.async_copy(ones_v, acc_sh.at[idx_v.at[k]], sem, add=True)
        return carry

    lax.fori_loop(0, _KCH, _scat, 0)

    def _drain(k, carry):
        pltpu.make_async_copy(ones_v, acc_sh.at[idx_v.at[k]], sem).wait()
        return carry

    lax.fori_loop(0, _KCH, _drain, 0)
    plsc.subcore_barrier()
    pltpu.sync_copy(acc_sh.at[pl.ds(s * _ZR, _ZR)],
                    out_hbm.at[c, pl.ds(s * _ZR, _ZR)])


@functools.partial(
    pl.kernel,
    mesh=_mesh,
    compiler_params=_sc_params,
    out_type=jax.ShapeDtypeStruct((_NC, _NPAD, _H), jnp.float32),
    scratch_types=[
        pltpu.VMEM((_KCH, _CH), jnp.int32),
        pltpu.VMEM((_KCH, _CH), jnp.int32),
        pltpu.VMEM((2, _CH, _H), jnp.float32),
        pltpu.VMEM((_ZR, _H), jnp.float32),
        pltpu.VMEM_SHARED((_NPAD, _H), jnp.float32),
        pltpu.SemaphoreType.DMA,
        pltpu.SemaphoreType.DMA,
    ],
)
def _sc_edge_sum(tab_hbm, src_hbm, dst_hbm, out_hbm,
                 src_v, dst_v, rows_v, zbuf_v, acc_sh, sem_g, sem_s):
    c = lax.axis_index("c")
    s = lax.axis_index("s")
    wid = s * _NC + c
    pltpu.sync_copy(src_hbm.at[wid], src_v)
    pltpu.sync_copy(dst_hbm.at[wid], dst_v)

    def _zero(i, carry):
        zbuf_v[i, :] = jnp.zeros((_H,), jnp.float32)
        return carry

    lax.fori_loop(0, _ZR, _zero, 0)
    pltpu.sync_copy(zbuf_v, acc_sh.at[pl.ds(s * _ZR, _ZR)])
    plsc.subcore_barrier()

    pltpu.async_copy(tab_hbm.at[src_v.at[0]], rows_v.at[0], sem_g)

    def _body(k, carry):
        b = lax.rem(k, 2)

        @pl.when(k >= 1)
        def _wait_prev_scatter():
            pltpu.make_async_copy(rows_v.at[1 - b],
                                  acc_sh.at[dst_v.at[k - 1]], sem_s).wait()

        @pl.when(k + 1 < _KCH)
        def _issue_next_gather():
            pltpu.async_copy(tab_hbm.at[src_v.at[k + 1]],
                             rows_v.at[1 - b], sem_g)

        pltpu.make_async_copy(tab_hbm.at[src_v.at[k]],
                              rows_v.at[b], sem_g).wait()
        pltpu.async_copy(rows_v.at[b], acc_sh.at[dst_v.at[k]], sem_s,
                         add=True)
        return carry

    lax.fori_loop(0, _KCH, _body, 0)
    pltpu.make_async_copy(rows_v.at[(_KCH - 1) % 2],
                          acc_sh.at[dst_v.at[_KCH - 1]], sem_s).wait()
    plsc.subcore_barrier()
    pltpu.sync_copy(acc_sh.at[pl.ds(s * _ZR, _ZR)],
                    out_hbm.at[c, pl.ds(s * _ZR, _ZR)])


def _tc1_body(x_ref, w_ref, d0_ref, d1_ref, xs_ref, dinv_ref):
    deg = d0_ref[...] + d1_ref[...] + 1.0
    dinv = lax.rsqrt(deg)
    xw = jnp.dot(x_ref[...], w_ref[...], preferred_element_type=jnp.float32)
    xs_ref[...] = xw * dinv
    dinv_ref[...] = dinv


def _tc2_body(s0_ref, s1_ref, xs_ref, dinv_ref, b_ref, w_ref, out_ref):
    dinv = dinv_ref[...]
    h = dinv * (s0_ref[...] + s1_ref[...] + xs_ref[...]) + b_ref[...]
    h = jnp.maximum(h, 0.0)
    out_ref[...] = jnp.dot(h, w_ref[...],
                           preferred_element_type=jnp.float32) * dinv


def _tc3_body(s0_ref, s1_ref, xs_ref, dinv_ref, b_ref, w_ref, bf_ref, out_ref):
    dinv = dinv_ref[...]
    h = dinv * (s0_ref[...] + s1_ref[...] + xs_ref[...]) + b_ref[...]
    h = jnp.maximum(h, 0.0)
    out_ref[...] = jnp.dot(h, w_ref[...],
                           preferred_element_type=jnp.float32) + bf_ref[...]


_tc1 = pl.pallas_call(
    _tc1_body,
    out_shape=(jax.ShapeDtypeStruct((_N, _H), jnp.float32),
               jax.ShapeDtypeStruct((_N, 1), jnp.float32)),
)

_tc2 = pl.pallas_call(
    _tc2_body,
    out_shape=jax.ShapeDtypeStruct((_N, _H), jnp.float32),
)

_tc3 = pl.pallas_call(
    _tc3_body,
    out_shape=jax.ShapeDtypeStruct((_N, 1), jnp.float32),
)


def kernel(x, edge_index, W1, b1, W2, b2, Wfc, bfc):
    ei = edge_index.astype(jnp.int32)
    ppw = _EPW - _E // _NW
    srcm = ei[0].reshape(_NW, _E // _NW)
    dstm = ei[1].reshape(_NW, _E // _NW)
    pad_src = jnp.zeros((_NW, ppw), jnp.int32)
    pad_dst = jnp.broadcast_to(
        _N + (jnp.arange(ppw, dtype=jnp.int32) % (_NPAD - _N)), (_NW, ppw))
    src_p = jnp.concatenate([srcm, pad_src], axis=1).reshape(_NW, _KCH, _CH)
    dst_p = jnp.concatenate([dstm, pad_dst], axis=1).reshape(_NW, _KCH, _CH)

    dega = _sc_deg(dst_p)
    d0 = dega[0, :_N, 0:1]
    d1 = dega[1, :_N, 0:1]

    xs1, dinv = _tc1(x, W1, d0, d1)
    s1 = _sc_edge_sum(xs1, src_p, dst_p)
    xs2 = _tc2(s1[0, :_N], s1[1, :_N], xs1, dinv, b1.reshape(1, _H), W2)
    s2 = _sc_edge_sum(xs2, src_p, dst_p)
    out = _tc3(s2[0, :_N], s2[1, :_N], xs2, dinv, b2.reshape(1, _H),
               Wfc, bfc.reshape(1, 1))
    return out


# speedup vs baseline: 1.4522x; 1.4077x over previous
import functools

import jax
import jax.numpy as jnp
from jax import lax
from jax.experimental import pallas as pl
from jax.experimental.pallas import tpu as pltpu, tpu_sc as plsc

_N = 10000
_E = 320000
_D = 128
_H = 16

_NC = 2
_NS = 16
_NW = _NC * _NS
_CH = 512
_KCH = 20
_EPW = _KCH * _CH
_PADE = _NW * _EPW - _E
_NPAD = 10112
_ZR = _NPAD // _NS

_mesh = plsc.VectorSubcoreMesh(core_axis_name="c", subcore_axis_name="s")
_sc_params = pltpu.CompilerParams(use_tc_tiling_on_sc=False)


@functools.partial(
    pl.kernel,
    mesh=_mesh,
    compiler_params=_sc_params,
    out_type=jax.ShapeDtypeStruct((_NC, _NPAD, _H), jnp.float32),
    scratch_types=[
        pltpu.VMEM((_KCH, _CH), jnp.int32),
        pltpu.VMEM((_CH, _H), jnp.float32),
        pltpu.VMEM((_ZR, _H), jnp.float32),
        pltpu.VMEM_SHARED((_NPAD, _H), jnp.float32),
        pltpu.SemaphoreType.DMA,
    ],
)
def _sc_deg(dst_hbm, out_hbm, idx_v, ones_v, zbuf_v, acc_sh, sem):
    c = lax.axis_index("c")
    s = lax.axis_index("s")
    wid = s * _NC + c
    pltpu.sync_copy(dst_hbm.at[wid], idx_v)

    def _fill(i, carry):
        ones_v[i, :] = jnp.full((_H,), 1.0, jnp.float32)
        return carry

    lax.fori_loop(0, _CH, _fill, 0)

    def _zero(i, carry):
        zbuf_v[i, :] = jnp.zeros((_H,), jnp.float32)
        return carry

    lax.fori_loop(0, _ZR, _zero, 0)
    pltpu.sync_copy(zbuf_v, acc_sh.at[pl.ds(s * _ZR, _ZR)])
    plsc.subcore_barrier()

    def _scat(k, carry):
        pltpu.async_copy(ones_v, acc_sh.at[idx_v.at[k]], sem, add=True)
        return carry

    lax.fori_loop(0, _KCH, _scat, 0)

    def _drain(k, carry):
        pltpu.make_async_copy(ones_v, acc_sh.at[idx_v.at[k]], sem).wait()
        return carry

    lax.fori_loop(0, _KCH, _drain, 0)
    plsc.subcore_barrier()
    pltpu.sync_copy(acc_sh.at[pl.ds(s * _ZR, _ZR)],
                    out_hbm.at[c, pl.ds(s * _ZR, _ZR)])


@functools.partial(
    pl.kernel,
    mesh=_mesh,
    compiler_params=_sc_params,
    out_type=jax.ShapeDtypeStruct((_NC, _NPAD, _H), jnp.float32),
    scratch_types=[
        pltpu.VMEM((_KCH, _CH), jnp.int32),
        pltpu.VMEM((_KCH, _CH), jnp.int32),
        pltpu.VMEM((2, _CH, _H), jnp.float32),
        pltpu.VMEM((_ZR, _H), jnp.float32),
        pltpu.VMEM_SHARED((_NPAD, _H), jnp.float32),
        pltpu.VMEM_SHARED((_NPAD, _H), jnp.float32),
        pltpu.SemaphoreType.DMA,
        pltpu.SemaphoreType.DMA,
    ],
)
def _sc_edge_sum(tab_hbm, src_hbm, dst_hbm, out_hbm,
                 src_v, dst_v, rows_v, zbuf_v, acc_sh, tab_sh, sem_g, sem_s):
    c = lax.axis_index("c")
    s = lax.axis_index("s")
    wid = s * _NC + c
    pltpu.sync_copy(src_hbm.at[wid], src_v)
    pltpu.sync_copy(dst_hbm.at[wid], dst_v)

    @pl.when(s < _NS - 1)
    def _stage_full():
        pltpu.sync_copy(tab_hbm.at[pl.ds(s * _ZR, _ZR)],
                        tab_sh.at[pl.ds(s * _ZR, _ZR)])

    @pl.when(s == _NS - 1)
    def _stage_tail():
        pltpu.sync_copy(tab_hbm.at[pl.ds((_NS - 1) * _ZR, _N - (_NS - 1) * _ZR)],
                        tab_sh.at[pl.ds((_NS - 1) * _ZR, _N - (_NS - 1) * _ZR)])

    def _zero(i, carry):
        zbuf_v[i, :] = jnp.zeros((_H,), jnp.float32)
        return carry

    lax.fori_loop(0, _ZR, _zero, 0)
    pltpu.sync_copy(zbuf_v, acc_sh.at[pl.ds(s * _ZR, _ZR)])
    plsc.subcore_barrier()

    pltpu.async_copy(tab_sh.at[src_v.at[0]], rows_v.at[0], sem_g)

    def _body(k, carry):
        b = lax.rem(k, 2)

        @pl.when(k >= 1)
        def _wait_prev_scatter():
            pltpu.make_async_copy(rows_v.at[1 - b],
                                  acc_sh.at[dst_v.at[k - 1]], sem_s).wait()

        @pl.when(k + 1 < _KCH)
        def _issue_next_gather():
            pltpu.async_copy(tab_sh.at[src_v.at[k + 1]],
                             rows_v.at[1 - b], sem_g)

        pltpu.make_async_copy(tab_sh.at[src_v.at[k]],
                              rows_v.at[b], sem_g).wait()
        pltpu.async_copy(rows_v.at[b], acc_sh.at[dst_v.at[k]], sem_s,
                         add=True)
        return carry

    lax.fori_loop(0, _KCH, _body, 0)
    pltpu.make_async_copy(rows_v.at[(_KCH - 1) % 2],
                          acc_sh.at[dst_v.at[_KCH - 1]], sem_s).wait()
    plsc.subcore_barrier()
    pltpu.sync_copy(acc_sh.at[pl.ds(s * _ZR, _ZR)],
                    out_hbm.at[c, pl.ds(s * _ZR, _ZR)])


def _tc1_body(x_ref, w_ref, d0_ref, d1_ref, xs_ref, dinv_ref):
    deg = d0_ref[...] + d1_ref[...] + 1.0
    dinv = lax.rsqrt(deg)
    xw = jnp.dot(x_ref[...], w_ref[...], preferred_element_type=jnp.float32)
    xs_ref[...] = xw * dinv
    dinv_ref[...] = dinv


def _tc2_body(s0_ref, s1_ref, xs_ref, dinv_ref, b_ref, w_ref, out_ref):
    dinv = dinv_ref[...]
    h = dinv * (s0_ref[...] + s1_ref[...] + xs_ref[...]) + b_ref[...]
    h = jnp.maximum(h, 0.0)
    out_ref[...] = jnp.dot(h, w_ref[...],
                           preferred_element_type=jnp.float32) * dinv


def _tc3_body(s0_ref, s1_ref, xs_ref, dinv_ref, b_ref, w_ref, bf_ref, out_ref):
    dinv = dinv_ref[...]
    h = dinv * (s0_ref[...] + s1_ref[...] + xs_ref[...]) + b_ref[...]
    h = jnp.maximum(h, 0.0)
    out_ref[...] = jnp.dot(h, w_ref[...],
                           preferred_element_type=jnp.float32) + bf_ref[...]


_tc1 = pl.pallas_call(
    _tc1_body,
    out_shape=(jax.ShapeDtypeStruct((_N, _H), jnp.float32),
               jax.ShapeDtypeStruct((_N, 1), jnp.float32)),
)

_tc2 = pl.pallas_call(
    _tc2_body,
    out_shape=jax.ShapeDtypeStruct((_N, _H), jnp.float32),
)

_tc3 = pl.pallas_call(
    _tc3_body,
    out_shape=jax.ShapeDtypeStruct((_N, 1), jnp.float32),
)


def kernel(x, edge_index, W1, b1, W2, b2, Wfc, bfc):
    ei = edge_index.astype(jnp.int32)
    ppw = _EPW - _E // _NW
    srcm = ei[0].reshape(_NW, _E // _NW)
    dstm = ei[1].reshape(_NW, _E // _NW)
    pad_src = jnp.zeros((_NW, ppw), jnp.int32)
    pad_dst = jnp.broadcast_to(
        _N + (jnp.arange(ppw, dtype=jnp.int32) % (_NPAD - _N)), (_NW, ppw))
    src_p = jnp.concatenate([srcm, pad_src], axis=1).reshape(_NW, _KCH, _CH)
    dst_p = jnp.concatenate([dstm, pad_dst], axis=1).reshape(_NW, _KCH, _CH)

    dega = _sc_deg(dst_p)
    d0 = dega[0, :_N, 0:1]
    d1 = dega[1, :_N, 0:1]

    xs1, dinv = _tc1(x, W1, d0, d1)
    s1 = _sc_edge_sum(xs1, src_p, dst_p)
    xs2 = _tc2(s1[0, :_N], s1[1, :_N], xs1, dinv, b1.reshape(1, _H), W2)
    s2 = _sc_edge_sum(xs2, src_p, dst_p)
    out = _tc3(s2[0, :_N], s2[1, :_N], xs2, dinv, b2.reshape(1, _H),
               Wfc, bfc.reshape(1, 1))
    return out


# speedup vs baseline: 1.6340x; 1.1252x over previous
import functools

import jax
import jax.numpy as jnp
from jax import lax
from jax.experimental import pallas as pl
from jax.experimental.pallas import tpu as pltpu, tpu_sc as plsc

_N = 10000
_E = 320000
_D = 128
_H = 16

_NC = 2
_NS = 16
_NW = _NC * _NS
_CH = 512
_KCH = 20
_EPW = _KCH * _CH
_PADE = _NW * _EPW - _E
_NPAD = 10112
_ZR = _NPAD // _NS

_mesh = plsc.VectorSubcoreMesh(core_axis_name="c", subcore_axis_name="s")
_sc_params = pltpu.CompilerParams(use_tc_tiling_on_sc=False)


@functools.partial(
    pl.kernel,
    mesh=_mesh,
    compiler_params=_sc_params,
    out_type=jax.ShapeDtypeStruct((_NC, _NPAD, _H), jnp.float32),
    scratch_types=[
        pltpu.VMEM((_KCH, _CH), jnp.int32),
        pltpu.VMEM((_CH, _H), jnp.float32),
        pltpu.VMEM((_ZR, _H), jnp.float32),
        pltpu.VMEM_SHARED((_NPAD, _H), jnp.float32),
        pltpu.SemaphoreType.DMA,
    ],
)
def _sc_deg(dst_hbm, out_hbm, idx_v, ones_v, zbuf_v, acc_sh, sem):
    c = lax.axis_index("c")
    s = lax.axis_index("s")
    wid = s * _NC + c
    pltpu.sync_copy(dst_hbm.at[wid], idx_v)

    def _fill(i, carry):
        ones_v[i, :] = jnp.full((_H,), 1.0, jnp.float32)
        return carry

    lax.fori_loop(0, _CH, _fill, 0)

    def _zero(i, carry):
        zbuf_v[i, :] = jnp.zeros((_H,), jnp.float32)
        return carry

    lax.fori_loop(0, _ZR, _zero, 0)
    pltpu.sync_copy(zbuf_v, acc_sh.at[pl.ds(s * _ZR, _ZR)])
    plsc.subcore_barrier()

    def _scat(k, carry):
        pltpu.async_copy(ones_v, acc_sh.at[idx_v.at[k]], sem, add=True)
        return carry

    lax.fori_loop(0, _KCH, _scat, 0)

    def _drain(k, carry):
        pltpu.make_async_copy(ones_v, acc_sh.at[idx_v.at[k]], sem).wait()
        return carry

    lax.fori_loop(0, _KCH, _drain, 0)
    plsc.subcore_barrier()
    pltpu.sync_copy(acc_sh.at[pl.ds(s * _ZR, _ZR)],
                    out_hbm.at[c, pl.ds(s * _ZR, _ZR)])


@functools.partial(
    pl.kernel,
    mesh=_mesh,
    compiler_params=_sc_params,
    out_type=jax.ShapeDtypeStruct((_NC, _NPAD, _H), jnp.float32),
    scratch_types=[
        pltpu.VMEM((_KCH, _CH), jnp.int32),
        pltpu.VMEM((_KCH, _CH), jnp.int32),
        pltpu.VMEM((2, _CH, _H), jnp.float32),
        pltpu.VMEM((_ZR, _H), jnp.float32),
        pltpu.VMEM_SHARED((_NPAD, _H), jnp.float32),
        pltpu.VMEM_SHARED((_NPAD, _H), jnp.float32),
        pltpu.SemaphoreType.DMA,
        pltpu.SemaphoreType.DMA,
    ],
)
def _sc_edge_sum(tab_hbm, src_hbm, dst_hbm, out_hbm,
                 src_v, dst_v, rows_v, zbuf_v, acc_sh, tab_sh, sem_g, sem_s):
    c = lax.axis_index("c")
    s = lax.axis_index("s")
    wid = s * _NC + c
    pltpu.sync_copy(src_hbm.at[wid], src_v)
    pltpu.sync_copy(dst_hbm.at[wid], dst_v)

    @pl.when(s < _NS - 1)
    def _stage_full():
        pltpu.sync_copy(tab_hbm.at[pl.ds(s * _ZR, _ZR)],
                        tab_sh.at[pl.ds(s * _ZR, _ZR)])

    @pl.when(s == _NS - 1)
    def _stage_tail():
        pltpu.sync_copy(tab_hbm.at[pl.ds((_NS - 1) * _ZR, _N - (_NS - 1) * _ZR)],
                        tab_sh.at[pl.ds((_NS - 1) * _ZR, _N - (_NS - 1) * _ZR)])

    def _zero(i, carry):
        zbuf_v[i, :] = jnp.zeros((_H,), jnp.float32)
        return carry

    lax.fori_loop(0, _ZR, _zero, 0)
    pltpu.sync_copy(zbuf_v, acc_sh.at[pl.ds(s * _ZR, _ZR)])
    plsc.subcore_barrier()

    pltpu.async_copy(tab_sh.at[src_v.at[0]], rows_v.at[0], sem_g)

    def _body(k, carry):
        b = lax.rem(k, 2)

        @pl.when(k >= 1)
        def _wait_prev_scatter():
            pltpu.make_async_copy(rows_v.at[1 - b],
                                  acc_sh.at[dst_v.at[k - 1]], sem_s).wait()

        @pl.when(k + 1 < _KCH)
        def _issue_next_gather():
            pltpu.async_copy(tab_sh.at[src_v.at[k + 1]],
                             rows_v.at[1 - b], sem_g)

        pltpu.make_async_copy(tab_sh.at[src_v.at[k]],
                              rows_v.at[b], sem_g).wait()
        pltpu.async_copy(rows_v.at[b], acc_sh.at[dst_v.at[k]], sem_s,
                         add=True)
        return carry

    lax.fori_loop(0, _KCH, _body, 0)
    pltpu.make_async_copy(rows_v.at[(_KCH - 1) % 2],
                          acc_sh.at[dst_v.at[_KCH - 1]], sem_s).wait()
    plsc.subcore_barrier()
    pltpu.sync_copy(acc_sh.at[pl.ds(s * _ZR, _ZR)],
                    out_hbm.at[c, pl.ds(s * _ZR, _ZR)])


def _tc1_body(x_ref, w_ref, dega_ref, xs_ref, dinv_ref):
    deg = dega_ref[0, :_N, :] + dega_ref[1, :_N, :] + 1.0
    dinv = lax.rsqrt(deg)
    xw = jnp.dot(x_ref[...], w_ref[...], preferred_element_type=jnp.float32)
    xs_ref[...] = xw * dinv
    dinv_ref[...] = dinv


def _tc2_body(s_ref, xs_ref, dinv_ref, b_ref, w_ref, out_ref):
    dinv = dinv_ref[...]
    h = dinv * (s_ref[0, :_N, :] + s_ref[1, :_N, :] + xs_ref[...]) + b_ref[...]
    h = jnp.maximum(h, 0.0)
    out_ref[...] = jnp.dot(h, w_ref[...],
                           preferred_element_type=jnp.float32) * dinv


def _tc3_body(s_ref, xs_ref, dinv_ref, b_ref, w_ref, bf_ref, out_ref):
    dinv = dinv_ref[...]
    h = dinv * (s_ref[0, :_N, :] + s_ref[1, :_N, :] + xs_ref[...]) + b_ref[...]
    h = jnp.maximum(h, 0.0)
    out_ref[...] = jnp.dot(h, w_ref[...],
                           preferred_element_type=jnp.float32) + bf_ref[...]


_tc1 = pl.pallas_call(
    _tc1_body,
    out_shape=(jax.ShapeDtypeStruct((_N, _H), jnp.float32),
               jax.ShapeDtypeStruct((_N, _H), jnp.float32)),
)

_tc2 = pl.pallas_call(
    _tc2_body,
    out_shape=jax.ShapeDtypeStruct((_N, _H), jnp.float32),
)

_tc3 = pl.pallas_call(
    _tc3_body,
    out_shape=jax.ShapeDtypeStruct((_N, 1), jnp.float32),
)


def kernel(x, edge_index, W1, b1, W2, b2, Wfc, bfc):
    ei = edge_index.astype(jnp.int32)
    ppw = _EPW - _E // _NW
    srcm = ei[0].reshape(_NW, _E // _NW)
    dstm = ei[1].reshape(_NW, _E // _NW)
    pad_src = jnp.zeros((_NW, ppw), jnp.int32)
    pad_dst = jnp.broadcast_to(
        _N + (jnp.arange(ppw, dtype=jnp.int32) % (_NPAD - _N)), (_NW, ppw))
    src_p = jnp.concatenate([srcm, pad_src], axis=1).reshape(_NW, _KCH, _CH)
    dst_p = jnp.concatenate([dstm, pad_dst], axis=1).reshape(_NW, _KCH, _CH)

    dega = _sc_deg(dst_p)
    xs1, dinv = _tc1(x, W1, dega)
    s1 = _sc_edge_sum(xs1, src_p, dst_p)
    xs2 = _tc2(s1, xs1, dinv, b1.reshape(1, _H), W2)
    s2 = _sc_edge_sum(xs2, src_p, dst_p)
    out = _tc3(s2, xs2, dinv, b2.reshape(1, _H), Wfc, bfc.reshape(1, 1))
    return out


# speedup vs baseline: 1.7529x; 1.0728x over previous
import functools

import jax
import jax.numpy as jnp
from jax import lax
from jax.experimental import pallas as pl
from jax.experimental.pallas import tpu as pltpu, tpu_sc as plsc

_N = 10000
_E = 320000
_D = 128
_H = 16

_NC = 2
_NS = 16
_NW = _NC * _NS
_CH = 512
_KCH = 20
_EPW = _KCH * _CH
_RPW = _E // _NW
_TAIL = _RPW - (_KCH - 1) * _CH
_NPAD = 10112
_ZR = _NPAD // _NS

_mesh = plsc.VectorSubcoreMesh(core_axis_name="c", subcore_axis_name="s")
_sc_params = pltpu.CompilerParams(use_tc_tiling_on_sc=False)


def _stage_idx(ei_hbm, row, base, idx_v, fill_vec, sem):
    def _cp(k, carry):
        pltpu.async_copy(ei_hbm.at[row, pl.ds(base + k * _CH, _CH)],
                         idx_v.at[k], sem)
        return carry

    lax.fori_loop(0, _KCH - 1, _cp, 0)
    pltpu.async_copy(ei_hbm.at[row, pl.ds(base + (_KCH - 1) * _CH, _TAIL)],
                     idx_v.at[_KCH - 1, pl.ds(0, _TAIL)], sem)

    def _fillk(i, carry):
        idx_v[_KCH - 1, pl.ds(_TAIL + i * 16, 16)] = fill_vec
        return carry

    lax.fori_loop(0, (_CH - _TAIL) // 16, _fillk, 0)

    def _dr(k, carry):
        pltpu.make_async_copy(ei_hbm.at[row, pl.ds(base + k * _CH, _CH)],
                              idx_v.at[k], sem).wait()
        return carry

    lax.fori_loop(0, _KCH - 1, _dr, 0)
    pltpu.make_async_copy(
        ei_hbm.at[row, pl.ds(base + (_KCH - 1) * _CH, _TAIL)],
        idx_v.at[_KCH - 1, pl.ds(0, _TAIL)], sem).wait()


@functools.partial(
    pl.kernel,
    mesh=_mesh,
    compiler_params=_sc_params,
    out_type=jax.ShapeDtypeStruct((_NC, _NPAD, _H), jnp.float32),
    scratch_types=[
        pltpu.VMEM((_KCH, _CH), jnp.int32),
        pltpu.VMEM((_CH, _H), jnp.float32),
        pltpu.VMEM((_ZR, _H), jnp.float32),
        pltpu.VMEM_SHARED((_NPAD, _H), jnp.float32),
        pltpu.SemaphoreType.DMA,
    ],
)
def _sc_deg(ei_hbm, out_hbm, idx_v, ones_v, zbuf_v, acc_sh, sem):
    c = lax.axis_index("c")
    s = lax.axis_index("s")
    wid = s * _NC + c
    _stage_idx(ei_hbm, 1, wid * _RPW, idx_v,
               _N + lax.iota(jnp.int32, 16), sem)

    def _fill(i, carry):
        ones_v[i, :] = jnp.full((_H,), 1.0, jnp.float32)
        return carry

    lax.fori_loop(0, _CH, _fill, 0)

    def _zero(i, carry):
        zbuf_v[i, :] = jnp.zeros((_H,), jnp.float32)
        return carry

    lax.fori_loop(0, _ZR, _zero, 0)
    pltpu.sync_copy(zbuf_v, acc_sh.at[pl.ds(s * _ZR, _ZR)])
    plsc.subcore_barrier()

    def _scat(k, carry):
        pltpu.async_copy(ones_v, acc_sh.at[idx_v.at[k]], sem, add=True)
        return carry

    lax.fori_loop(0, _KCH, _scat, 0)

    def _drain(k, carry):
        pltpu.make_async_copy(ones_v, acc_sh.at[idx_v.at[k]], sem).wait()
        return carry

    lax.fori_loop(0, _KCH, _drain, 0)
    plsc.subcore_barrier()
    pltpu.sync_copy(acc_sh.at[pl.ds(s * _ZR, _ZR)],
                    out_hbm.at[c, pl.ds(s * _ZR, _ZR)])


@functools.partial(
    pl.kernel,
    mesh=_mesh,
    compiler_params=_sc_params,
    out_type=jax.ShapeDtypeStruct((_NC, _NPAD, _H), jnp.float32),
    scratch_types=[
        pltpu.VMEM((_KCH, _CH), jnp.int32),
        pltpu.VMEM((_KCH, _CH), jnp.int32),
        pltpu.VMEM((2, _CH, _H), jnp.float32),
        pltpu.VMEM((_ZR, _H), jnp.float32),
        pltpu.VMEM_SHARED((_NPAD, _H), jnp.float32),
        pltpu.VMEM_SHARED((_NPAD, _H), jnp.float32),
        pltpu.SemaphoreType.DMA,
        pltpu.SemaphoreType.DMA,
    ],
)
def _sc_edge_sum(tab_hbm, ei_hbm, out_hbm,
                 src_v, dst_v, rows_v, zbuf_v, acc_sh, tab_sh, sem_g, sem_s):
    c = lax.axis_index("c")
    s = lax.axis_index("s")
    wid = s * _NC + c
    _stage_idx(ei_hbm, 0, wid * _RPW, src_v, jnp.zeros((16,), jnp.int32),
               sem_g)
    _stage_idx(ei_hbm, 1, wid * _RPW, dst_v,
               _N + lax.iota(jnp.int32, 16), sem_s)

    @pl.when(s < _NS - 1)
    def _stage_full():
        pltpu.sync_copy(tab_hbm.at[pl.ds(s * _ZR, _ZR)],
                        tab_sh.at[pl.ds(s * _ZR, _ZR)])

    @pl.when(s == _NS - 1)
    def _stage_tail():
        pltpu.sync_copy(tab_hbm.at[pl.ds((_NS - 1) * _ZR, _N - (_NS - 1) * _ZR)],
                        tab_sh.at[pl.ds((_NS - 1) * _ZR, _N - (_NS - 1) * _ZR)])

    def _zero(i, carry):
        zbuf_v[i, :] = jnp.zeros((_H,), jnp.float32)
        return carry

    lax.fori_loop(0, _ZR, _zero, 0)
    pltpu.sync_copy(zbuf_v, acc_sh.at[pl.ds(s * _ZR, _ZR)])
    plsc.subcore_barrier()

    pltpu.async_copy(tab_sh.at[src_v.at[0]], rows_v.at[0], sem_g)

    def _body(k, carry):
        b = lax.rem(k, 2)

        @pl.when(k >= 1)
        def _wait_prev_scatter():
            pltpu.make_async_copy(rows_v.at[1 - b],
                                  acc_sh.at[dst_v.at[k - 1]], sem_s).wait()

        @pl.when(k + 1 < _KCH)
        def _issue_next_gather():
            pltpu.async_copy(tab_sh.at[src_v.at[k + 1]],
                             rows_v.at[1 - b], sem_g)

        pltpu.make_async_copy(tab_sh.at[src_v.at[k]],
                              rows_v.at[b], sem_g).wait()
        pltpu.async_copy(rows_v.at[b], acc_sh.at[dst_v.at[k]], sem_s,
                         add=True)
        return carry

    lax.fori_loop(0, _KCH, _body, 0)
    pltpu.make_async_copy(rows_v.at[(_KCH - 1) % 2],
                          acc_sh.at[dst_v.at[_KCH - 1]], sem_s).wait()
    plsc.subcore_barrier()
    pltpu.sync_copy(acc_sh.at[pl.ds(s * _ZR, _ZR)],
                    out_hbm.at[c, pl.ds(s * _ZR, _ZR)])


def _tc1_body(x_ref, w_ref, dega_ref, xs_ref, dinv_ref):
    deg = dega_ref[0, :_N, :] + dega_ref[1, :_N, :] + 1.0
    dinv = lax.rsqrt(deg)
    xw = jnp.dot(x_ref[...], w_ref[...], preferred_element_type=jnp.float32)
    xs_ref[...] = xw * dinv
    dinv_ref[...] = dinv


def _tc2_body(s_ref, xs_ref, dinv_ref, b_ref, w_ref, out_ref):
    dinv = dinv_ref[...]
    h = dinv * (s_ref[0, :_N, :] + s_ref[1, :_N, :] + xs_ref[...]) + b_ref[...]
    h = jnp.maximum(h, 0.0)
    out_ref[...] = jnp.dot(h, w_ref[...],
                           preferred_element_type=jnp.float32) * dinv


def _tc3_body(s_ref, xs_ref, dinv_ref, b_ref, w_ref, bf_ref, out_ref):
    dinv = dinv_ref[...]
    h = dinv * (s_ref[0, :_N, :] + s_ref[1, :_N, :] + xs_ref[...]) + b_ref[...]
    h = jnp.maximum(h, 0.0)
    out_ref[...] = jnp.dot(h, w_ref[...],
                           preferred_element_type=jnp.float32) + bf_ref[...]


_tc1 = pl.pallas_call(
    _tc1_body,
    out_shape=(jax.ShapeDtypeStruct((_N, _H), jnp.float32),
               jax.ShapeDtypeStruct((_N, _H), jnp.float32)),
)

_tc2 = pl.pallas_call(
    _tc2_body,
    out_shape=jax.ShapeDtypeStruct((_N, _H), jnp.float32),
)

_tc3 = pl.pallas_call(
    _tc3_body,
    out_shape=jax.ShapeDtypeStruct((_N, 1), jnp.float32),
)


def kernel(x, edge_index, W1, b1, W2, b2, Wfc, bfc):
    ei = edge_index.astype(jnp.int32)
    dega = _sc_deg(ei)
    xs1, dinv = _tc1(x, W1, dega)
    s1 = _sc_edge_sum(xs1, ei)
    xs2 = _tc2(s1, xs1, dinv, b1.reshape(1, _H), W2)
    s2 = _sc_edge_sum(xs2, ei)
    out = _tc3(s2, xs2, dinv, b2.reshape(1, _H), Wfc, bfc.reshape(1, 1))
    return out


# speedup vs baseline: 2.6119x; 1.4900x over previous
import functools

import jax
import jax.numpy as jnp
from jax import lax
from jax.experimental import pallas as pl
from jax.experimental.pallas import tpu as pltpu, tpu_sc as plsc

_N = 10000
_E = 320000
_D = 128
_H = 16

_NC = 2
_NS = 16
_NW = _NC * _NS
_CH = 512
_KCH = 20
_EPW = _KCH * _CH
_RPW = _E // _NW
_TAIL = _RPW - (_KCH - 1) * _CH
_NPAD = 10112
_ZR = _NPAD // _NS
_NPACK = _NPAD * _H // 128
_NR = _N * _H // 128

_mesh = plsc.VectorSubcoreMesh(core_axis_name="c", subcore_axis_name="s")
_sc_params = pltpu.CompilerParams(use_tc_tiling_on_sc=False)


def _stage_idx(ei_hbm, row, base, idx_v, fill_vec, sem):
    def _cp(k, carry):
        pltpu.async_copy(ei_hbm.at[row, pl.ds(base + k * _CH, _CH)],
                         idx_v.at[k], sem)
        return carry

    lax.fori_loop(0, _KCH - 1, _cp, 0)
    pltpu.async_copy(ei_hbm.at[row, pl.ds(base + (_KCH - 1) * _CH, _TAIL)],
                     idx_v.at[_KCH - 1, pl.ds(0, _TAIL)], sem)

    def _fillk(i, carry):
        idx_v[_KCH - 1, pl.ds(_TAIL + i * 16, 16)] = fill_vec
        return carry

    lax.fori_loop(0, (_CH - _TAIL) // 16, _fillk, 0)

    def _dr(k, carry):
        pltpu.make_async_copy(ei_hbm.at[row, pl.ds(base + k * _CH, _CH)],
                              idx_v.at[k], sem).wait()
        return carry

    lax.fori_loop(0, _KCH - 1, _dr, 0)
    pltpu.make_async_copy(
        ei_hbm.at[row, pl.ds(base + (_KCH - 1) * _CH, _TAIL)],
        idx_v.at[_KCH - 1, pl.ds(0, _TAIL)], sem).wait()


@functools.partial(
    pl.kernel,
    mesh=_mesh,
    compiler_params=_sc_params,
    out_type=jax.ShapeDtypeStruct((_NC, _NPAD, _H), jnp.float32),
    scratch_types=[
        pltpu.VMEM((_KCH, _CH), jnp.int32),
        pltpu.VMEM((_CH, _H), jnp.float32),
        pltpu.VMEM((_ZR, _H), jnp.float32),
        pltpu.VMEM_SHARED((_NPAD, _H), jnp.float32),
        pltpu.SemaphoreType.DMA,
    ],
)
def _sc_deg(ei_hbm, out_hbm, idx_v, ones_v, zbuf_v, acc_sh, sem):
    c = lax.axis_index("c")
    s = lax.axis_index("s")
    wid = s * _NC + c
    _stage_idx(ei_hbm, 1, wid * _RPW, idx_v,
               _N + lax.iota(jnp.int32, 16), sem)

    def _fill(i, carry):
        ones_v[i, :] = jnp.full((_H,), 1.0, jnp.float32)
        return carry

    lax.fori_loop(0, _CH, _fill, 0)

    def _zero(i, carry):
        zbuf_v[i, :] = jnp.zeros((_H,), jnp.float32)
        return carry

    lax.fori_loop(0, _ZR, _zero, 0)
    pltpu.sync_copy(zbuf_v, acc_sh.at[pl.ds(s * _ZR, _ZR)])
    plsc.subcore_barrier()

    def _scat(k, carry):
        pltpu.async_copy(ones_v, acc_sh.at[idx_v.at[k]], sem, add=True)
        return carry

    lax.fori_loop(0, _KCH, _scat, 0)

    def _drain(k, carry):
        pltpu.make_async_copy(ones_v, acc_sh.at[idx_v.at[k]], sem).wait()
        return carry

    lax.fori_loop(0, _KCH, _drain, 0)
    plsc.subcore_barrier()
    pltpu.sync_copy(acc_sh.at[pl.ds(s * _ZR, _ZR)],
                    out_hbm.at[c, pl.ds(s * _ZR, _ZR)])


@functools.partial(
    pl.kernel,
    mesh=_mesh,
    compiler_params=_sc_params,
    out_type=jax.ShapeDtypeStruct((_NC, _NPAD, _H), jnp.float32),
    scratch_types=[
        pltpu.VMEM((_KCH, _CH), jnp.int32),
        pltpu.VMEM((_KCH, _CH), jnp.int32),
        pltpu.VMEM((2, _CH, _H), jnp.float32),
        pltpu.VMEM((_ZR, _H), jnp.float32),
        pltpu.VMEM_SHARED((_NPAD, _H), jnp.float32),
        pltpu.VMEM_SHARED((_NPAD, _H), jnp.float32),
        pltpu.SemaphoreType.DMA,
        pltpu.SemaphoreType.DMA,
    ],
)
def _sc_edge_sum(tab_hbm, ei_hbm, out_hbm,
                 src_v, dst_v, rows_v, zbuf_v, acc_sh, tab_sh, sem_g, sem_s):
    c = lax.axis_index("c")
    s = lax.axis_index("s")
    wid = s * _NC + c
    _stage_idx(ei_hbm, 0, wid * _RPW, src_v, jnp.zeros((16,), jnp.int32),
               sem_g)
    _stage_idx(ei_hbm, 1, wid * _RPW, dst_v,
               _N + lax.iota(jnp.int32, 16), sem_s)

    pltpu.sync_copy(tab_hbm.at[pl.ds(s * _ZR, _ZR)],
                    tab_sh.at[pl.ds(s * _ZR, _ZR)])

    def _zero(i, carry):
        zbuf_v[i, :] = jnp.zeros((_H,), jnp.float32)
        return carry

    lax.fori_loop(0, _ZR, _zero, 0)
    pltpu.sync_copy(zbuf_v, acc_sh.at[pl.ds(s * _ZR, _ZR)])
    plsc.subcore_barrier()

    pltpu.async_copy(tab_sh.at[src_v.at[0]], rows_v.at[0], sem_g)

    def _body(k, carry):
        b = lax.rem(k, 2)

        @pl.when(k >= 1)
        def _wait_prev_scatter():
            pltpu.make_async_copy(rows_v.at[1 - b],
                                  acc_sh.at[dst_v.at[k - 1]], sem_s).wait()

        @pl.when(k + 1 < _KCH)
        def _issue_next_gather():
            pltpu.async_copy(tab_sh.at[src_v.at[k + 1]],
                             rows_v.at[1 - b], sem_g)

        pltpu.make_async_copy(tab_sh.at[src_v.at[k]],
                              rows_v.at[b], sem_g).wait()
        pltpu.async_copy(rows_v.at[b], acc_sh.at[dst_v.at[k]], sem_s,
                         add=True)
        return carry

    lax.fori_loop(0, _KCH, _body, 0)
    pltpu.make_async_copy(rows_v.at[(_KCH - 1) % 2],
                          acc_sh.at[dst_v.at[_KCH - 1]], sem_s).wait()
    plsc.subcore_barrier()
    pltpu.sync_copy(acc_sh.at[pl.ds(s * _ZR, _ZR)],
                    out_hbm.at[c, pl.ds(s * _ZR, _ZR)])


def _tc1_body(x8_ref, w_ref, dega_ref, xs_ref, dinv_ref):
    deg = dega_ref[0] + dega_ref[1] + 1.0
    dinv = lax.rsqrt(deg)
    w = w_ref[...]
    xw = jnp.concatenate([
        jnp.dot(x8_ref[:, j, :], w, preferred_element_type=jnp.float32)
        for j in range(8)], axis=1)
    xw = jnp.concatenate(
        [xw, jnp.zeros((_NPACK - _NR, 128), jnp.float32)], axis=0)
    xs_ref[...] = xw * dinv
    dinv_ref[...] = dinv


def _block_diag_w(w_ref, ncols):
    wt = jnp.tile(w_ref[...], (8, 8))[:, :ncols]
    r = lax.broadcasted_iota(jnp.int32, (128, ncols), 0)
    c = lax.broadcasted_iota(jnp.int32, (128, ncols), 1)
    blk = c // 16 if ncols == 128 else c
    return jnp.where(r // 16 == blk, wt, 0.0)


def _tc2_body(s_ref, xs_ref, dinv_ref, b_ref, w_ref, out_ref):
    dinv = dinv_ref[...]
    h = dinv * (s_ref[0] + s_ref[1] + xs_ref[...]) + b_ref[...]
    h = jnp.maximum(h, 0.0)
    bd = _block_diag_w(w_ref, 128)
    out_ref[...] = jnp.dot(h, bd, preferred_element_type=jnp.float32) * dinv


def _tc3_body(s_ref, xs_ref, dinv_ref, b_ref, w_ref, bf_ref, out_ref):
    dinv = dinv_ref[...]
    h = dinv * (s_ref[0] + s_ref[1] + xs_ref[...]) + b_ref[...]
    h = jnp.maximum(h, 0.0)
    bd = _block_diag_w(w_ref, 8)
    out_ref[...] = jnp.dot(h, bd, preferred_element_type=jnp.float32) \
        + bf_ref[...]


_tc1 = pl.pallas_call(
    _tc1_body,
    out_shape=(jax.ShapeDtypeStruct((_NPACK, 128), jnp.float32),
               jax.ShapeDtypeStruct((_NPACK, 128), jnp.float32)),
)

_tc2 = pl.pallas_call(
    _tc2_body,
    out_shape=jax.ShapeDtypeStruct((_NPACK, 128), jnp.float32),
)

_tc3 = pl.pallas_call(
    _tc3_body,
    out_shape=jax.ShapeDtypeStruct((_NPACK, 8), jnp.float32),
)


def kernel(x, edge_index, W1, b1, W2, b2, Wfc, bfc):
    ei = edge_index.astype(jnp.int32)
    x8 = x.reshape(_NR, 8, _D)
    dega = _sc_deg(ei)
    xs1, dinv = _tc1(x8, W1, dega.reshape(_NC, _NPACK, 128))
    s1 = _sc_edge_sum(xs1.reshape(_NPAD, _H), ei)
    b1p = jnp.tile(b1.reshape(1, _H), (1, 8))
    xs2 = _tc2(s1.reshape(_NC, _NPACK, 128), xs1, dinv, b1p, W2)
    s2 = _sc_edge_sum(xs2.reshape(_NPAD, _H), ei)
    b2p = jnp.tile(b2.reshape(1, _H), (1, 8))
    o = _tc3(s2.reshape(_NC, _NPACK, 128), xs2, dinv, b2p, Wfc,
             bfc.reshape(1, 1))
    return o[:_NR].reshape(_N, 1)
